# baseline (speedup 1.0000x reference)
import jax
import jax.numpy as jnp
from jax.experimental import pallas as pl

_B, _Q, _D, _H, _R = 2, 2048, 768, 12, 12
_HD = _D // _H
_RHO = 2.0
_MAXOFF = 6.0
_W = 8


def _shift_cols(a, d):
    if d == 0:
        return a
    z = jnp.zeros((a.shape[0], abs(d)), a.dtype)
    if d > 0:
        return jnp.concatenate([a[:, d:], z], axis=1)
    return jnp.concatenate([z, a[:, :d]], axis=1)


def _qkv_body(x_ref, qw_ref, kw_ref, vw_ref, qb_ref, kb_ref, vb_ref,
              qf_ref, kf_ref, vf_ref):
    xb = x_ref[0]
    dn = (((1,), (1,)), ((), ()))
    qf_ref[0] = jax.lax.dot_general(
        qw_ref[...], xb, dn, preferred_element_type=jnp.float32) + qb_ref[...]
    kf_ref[0] = jax.lax.dot_general(
        kw_ref[...], xb, dn, preferred_element_type=jnp.float32) + kb_ref[...]
    vf_ref[0] = jax.lax.dot_general(
        vw_ref[...], xb, dn, preferred_element_type=jnp.float32) + vb_ref[...]


def _off_body(qf_ref, dw0_ref, dw1_ref, dw2_ref, dwb_ref, pw_ref, pwb_ref,
              off_ref):
    f = qf_ref[0]
    up = _shift_cols(f, -1)
    dn = _shift_cols(f, 1)
    dw = (dw0_ref[...] * up + dw1_ref[...] * f + dw2_ref[...] * dn
          + dwb_ref[...])
    g = 0.5 * dw * (1.0 + jax.lax.erf(dw * (2.0 ** -0.5)))
    raw = jnp.dot(pw_ref[...], g,
                  preferred_element_type=jnp.float32) + pwb_ref[...]
    off_ref[0] = jnp.tanh(raw) * _MAXOFF


def _attn_body(qf_ref, kf_ref, vf_ref, off_ref, anc_ref, rs_ref, out_ref):
    rs = rs_ref[0, 0]
    anc = anc_ref[...]
    qh = qf_ref[0]
    kh = kf_ref[0]
    vh = vf_ref[0]
    off = off_ref[0, 0]
    base = jax.lax.broadcasted_iota(jnp.int32, (_R, _Q), 1).astype(jnp.float32)
    pos = jnp.clip(base + anc + off, 0.0, float(_Q - 1))
    low = jnp.floor(pos)
    frac = pos - low
    delta = low - base
    dhi = jnp.ceil(pos) - base
    inv_sqrt_hd = 1.0 / (_HD ** 0.5)
    sels = []
    score = -rs * jnp.abs(pos - base)
    for d in range(-_W, _W + 1):
        df = float(d)
        sel = (jnp.where(delta == df, 1.0 - frac, 0.0)
               + jnp.where(dhi == df, frac, 0.0))
        sels.append(sel)
        s_d = jnp.sum(qh * _shift_cols(kh, d), axis=0,
                      keepdims=True) * inv_sqrt_hd
        score = score + s_d * sel
    m = jnp.max(score, axis=0, keepdims=True)
    e = jnp.exp(score - m)
    attn = e / jnp.sum(e, axis=0, keepdims=True)
    acc = jnp.zeros((_HD, _Q), jnp.float32)
    for i, d in enumerate(range(-_W, _W + 1)):
        w_d = jnp.sum(attn * sels[i], axis=0, keepdims=True)
        acc = acc + w_d * _shift_cols(vh, d)
    out_ref[0] = acc.astype(jnp.bfloat16)


def _oproj_body(a_ref, ow_ref, ob_ref, y_ref):
    dn = (((0,), (1,)), ((), ()))
    y_ref[0] = jax.lax.dot_general(
        a_ref[0], ow_ref[...], dn,
        preferred_element_type=jnp.float32) + ob_ref[...]


def kernel(x, qW, qB, kW, kB, vW, vB, oW, oB, dwW, dwB, pwW, pwB, rel_scale):
    b, q, d = x.shape
    f32 = jnp.float32
    bf16 = jnp.bfloat16
    tq = 512
    nq = q // tq

    xrow_blk = pl.BlockSpec((1, tq, d), lambda ib, iq: (ib, iq, 0))
    colt_blk = pl.BlockSpec((1, d, tq), lambda ib, iq: (ib, 0, iq))
    full_w = pl.BlockSpec((d, d), lambda ib, iq: (0, 0))
    colb = pl.BlockSpec((d, 1), lambda ib, iq: (0, 0))

    qft, kft, vft = pl.pallas_call(
        _qkv_body,
        grid=(b, nq),
        in_specs=[xrow_blk, full_w, full_w, full_w, colb, colb, colb],
        out_specs=(colt_blk, colt_blk, colt_blk),
        out_shape=(jax.ShapeDtypeStruct((b, d, q), f32),) * 3,
    )(x.astype(bf16), qW.astype(bf16), kW.astype(bf16), vW.astype(bf16),
      qB.reshape(d, 1), kB.reshape(d, 1), vB.reshape(d, 1))

    hr = _H * _R
    offt = pl.pallas_call(
        _off_body,
        grid=(b,),
        in_specs=[pl.BlockSpec((1, d, q), lambda i: (i, 0, 0)),
                  pl.BlockSpec((d, 1), lambda i: (0, 0)),
                  pl.BlockSpec((d, 1), lambda i: (0, 0)),
                  pl.BlockSpec((d, 1), lambda i: (0, 0)),
                  pl.BlockSpec((d, 1), lambda i: (0, 0)),
                  pl.BlockSpec((hr, d), lambda i: (0, 0)),
                  pl.BlockSpec((hr, 1), lambda i: (0, 0))],
        out_specs=pl.BlockSpec((1, hr, q), lambda i: (i, 0, 0)),
        out_shape=jax.ShapeDtypeStruct((b, hr, q), f32),
    )(qft, dwW[:, 0].reshape(d, 1), dwW[:, 1].reshape(d, 1),
      dwW[:, 2].reshape(d, 1), dwB.reshape(d, 1), pwW, pwB.reshape(hr, 1))

    off4 = offt.reshape(b, _H, _R, q)
    anchor = jnp.linspace(-_RHO, _RHO, _R).astype(f32).reshape(_R, 1)

    head_blk = pl.BlockSpec((1, _HD, q), lambda ib, ih: (ib, ih, 0))
    attn_t = pl.pallas_call(
        _attn_body,
        grid=(b, _H),
        in_specs=[head_blk, head_blk, head_blk,
                  pl.BlockSpec((1, 1, _R, q), lambda ib, ih: (ib, ih, 0, 0)),
                  pl.BlockSpec((_R, 1), lambda ib, ih: (0, 0)),
                  pl.BlockSpec((1, 1), lambda ib, ih: (0, 0))],
        out_specs=head_blk,
        out_shape=jax.ShapeDtypeStruct((b, d, q), bf16),
    )(qft, kft, vft, off4, anchor, jnp.asarray(rel_scale, f32).reshape(1, 1))

    y = pl.pallas_call(
        _oproj_body,
        grid=(b, nq),
        in_specs=[colt_blk, full_w,
                  pl.BlockSpec((1, d), lambda ib, iq: (0, 0))],
        out_specs=xrow_blk,
        out_shape=jax.ShapeDtypeStruct((b, q, d), f32),
    )(attn_t, oW.astype(bf16), oB.reshape(1, d))

    return y


# speedup vs baseline: 1.0602x; 1.0602x over previous
import jax
import jax.numpy as jnp
from jax.experimental import pallas as pl

_B, _Q, _D, _H, _R = 2, 2048, 768, 12, 12
_HD = _D // _H
_RHO = 2.0
_MAXOFF = 6.0
_W = 8


def _shift_cols(a, d):
    if d == 0:
        return a
    z = jnp.zeros((a.shape[0], abs(d)), a.dtype)
    if d > 0:
        return jnp.concatenate([a[:, d:], z], axis=1)
    return jnp.concatenate([z, a[:, :d]], axis=1)


def _qkv_body(x_ref, qw_ref, kw_ref, vw_ref, qb_ref, kb_ref, vb_ref,
              qf_ref, kf_ref, vf_ref):
    xb = x_ref[0]
    dn = (((1,), (1,)), ((), ()))
    qf_ref[0] = jax.lax.dot_general(
        qw_ref[...], xb, dn, preferred_element_type=jnp.float32) + qb_ref[...]
    kf_ref[0] = jax.lax.dot_general(
        kw_ref[...], xb, dn, preferred_element_type=jnp.float32) + kb_ref[...]
    vf_ref[0] = jax.lax.dot_general(
        vw_ref[...], xb, dn, preferred_element_type=jnp.float32) + vb_ref[...]


def _off_body(qf_ref, dw0_ref, dw1_ref, dw2_ref, dwb_ref, pw_ref, pwb_ref,
              off_ref):
    f = qf_ref[0]
    up = _shift_cols(f, -1)
    dn = _shift_cols(f, 1)
    dw = (dw0_ref[...] * up + dw1_ref[...] * f + dw2_ref[...] * dn
          + dwb_ref[...])
    g = 0.5 * dw * (1.0 + jax.lax.erf(dw * (2.0 ** -0.5)))
    raw = jnp.dot(pw_ref[...], g,
                  preferred_element_type=jnp.float32) + pwb_ref[...]
    off_ref[0] = jnp.tanh(raw) * _MAXOFF


def _attn_body(qf_ref, kf_ref, vf_ref, off_ref, anc_ref, rs_ref, out_ref):
    rs = rs_ref[0, 0]
    anc = anc_ref[...]
    qh = qf_ref[0]
    kh = kf_ref[0]
    vh = vf_ref[0]
    off = off_ref[0, 0]
    base = jax.lax.broadcasted_iota(jnp.int32, (_R, _Q), 1).astype(jnp.float32)
    pos = jnp.clip(base + anc + off, 0.0, float(_Q - 1))
    rel = pos - base
    qhs = qh * (1.0 / (_HD ** 0.5))
    sels = []
    score = -rs * jnp.abs(rel)
    for d in range(-_W, _W + 1):
        sel = jnp.maximum(0.0, 1.0 - jnp.abs(rel - float(d)))
        sels.append(sel)
        s_d = jnp.sum(qhs * _shift_cols(kh, d), axis=0,
                      keepdims=True)
        score = score + s_d * sel
    m = jnp.max(score, axis=0, keepdims=True)
    e = jnp.exp(score - m)
    attn = e / jnp.sum(e, axis=0, keepdims=True)
    acc = jnp.zeros((_HD, _Q), jnp.float32)
    for i, d in enumerate(range(-_W, _W + 1)):
        w_d = jnp.sum(attn * sels[i], axis=0, keepdims=True)
        acc = acc + w_d * _shift_cols(vh, d)
    out_ref[0] = acc


def _oproj_body(a_ref, ow_ref, ob_ref, y_ref):
    dn = (((0,), (1,)), ((), ()))
    y_ref[0] = jax.lax.dot_general(
        a_ref[0], ow_ref[...], dn,
        preferred_element_type=jnp.float32) + ob_ref[...]


def kernel(x, qW, qB, kW, kB, vW, vB, oW, oB, dwW, dwB, pwW, pwB, rel_scale):
    b, q, d = x.shape
    f32 = jnp.float32
    tq = 512
    nq = q // tq

    xrow_blk = pl.BlockSpec((1, tq, d), lambda ib, iq: (ib, iq, 0))
    colt_blk = pl.BlockSpec((1, d, tq), lambda ib, iq: (ib, 0, iq))
    full_w = pl.BlockSpec((d, d), lambda ib, iq: (0, 0))
    colb = pl.BlockSpec((d, 1), lambda ib, iq: (0, 0))

    qft, kft, vft = pl.pallas_call(
        _qkv_body,
        grid=(b, nq),
        in_specs=[xrow_blk, full_w, full_w, full_w, colb, colb, colb],
        out_specs=(colt_blk, colt_blk, colt_blk),
        out_shape=(jax.ShapeDtypeStruct((b, d, q), f32),) * 3,
    )(x, qW, kW, vW, qB.reshape(d, 1), kB.reshape(d, 1), vB.reshape(d, 1))

    hr = _H * _R
    offt = pl.pallas_call(
        _off_body,
        grid=(b,),
        in_specs=[pl.BlockSpec((1, d, q), lambda i: (i, 0, 0)),
                  pl.BlockSpec((d, 1), lambda i: (0, 0)),
                  pl.BlockSpec((d, 1), lambda i: (0, 0)),
                  pl.BlockSpec((d, 1), lambda i: (0, 0)),
                  pl.BlockSpec((d, 1), lambda i: (0, 0)),
                  pl.BlockSpec((hr, d), lambda i: (0, 0)),
                  pl.BlockSpec((hr, 1), lambda i: (0, 0))],
        out_specs=pl.BlockSpec((1, hr, q), lambda i: (i, 0, 0)),
        out_shape=jax.ShapeDtypeStruct((b, hr, q), f32),
    )(qft, dwW[:, 0].reshape(d, 1), dwW[:, 1].reshape(d, 1),
      dwW[:, 2].reshape(d, 1), dwB.reshape(d, 1), pwW, pwB.reshape(hr, 1))

    off4 = offt.reshape(b, _H, _R, q)
    anchor = jnp.linspace(-_RHO, _RHO, _R).astype(f32).reshape(_R, 1)

    head_blk = pl.BlockSpec((1, _HD, q), lambda ib, ih: (ib, ih, 0))
    attn_t = pl.pallas_call(
        _attn_body,
        grid=(b, _H),
        in_specs=[head_blk, head_blk, head_blk,
                  pl.BlockSpec((1, 1, _R, q), lambda ib, ih: (ib, ih, 0, 0)),
                  pl.BlockSpec((_R, 1), lambda ib, ih: (0, 0)),
                  pl.BlockSpec((1, 1), lambda ib, ih: (0, 0))],
        out_specs=head_blk,
        out_shape=jax.ShapeDtypeStruct((b, d, q), f32),
    )(qft, kft, vft, off4, anchor, jnp.asarray(rel_scale, f32).reshape(1, 1))

    y = pl.pallas_call(
        _oproj_body,
        grid=(b, nq),
        in_specs=[colt_blk, full_w,
                  pl.BlockSpec((1, d), lambda ib, iq: (0, 0))],
        out_specs=xrow_blk,
        out_shape=jax.ShapeDtypeStruct((b, q, d), f32),
    )(attn_t, oW, oB.reshape(1, d))

    return y
